# baseline (speedup 1.0000x reference)
import functools

import jax
import jax.numpy as jnp
from jax import lax
from jax.experimental import pallas as pl
from jax.experimental.pallas import tpu as pltpu
from jax.experimental.pallas import tpu_sc as plsc

_SHIFT = 1000


def kernel(x):
    B, S, T = x.shape
    R = B * S
    info = plsc.get_sparse_core_info()
    NC, NS = info.num_cores, info.num_subcores
    NW = NC * NS
    rows_per_w = R // NW

    mesh = plsc.VectorSubcoreMesh(core_axis_name="c", subcore_axis_name="s")

    @functools.partial(
        pl.kernel,
        mesh=mesh,
        out_type=jax.ShapeDtypeStruct((R, T), jnp.float32),
        compiler_params=pltpu.CompilerParams(use_tc_tiling_on_sc=False),
    )
    def roll_k(x_hbm, out_hbm):
        wid = lax.axis_index("s") * NC + lax.axis_index("c")
        r0 = wid * rows_per_w
        rows = pl.ds(r0, rows_per_w)
        pltpu.sync_copy(
            x_hbm.at[rows, pl.ds(0, T - _SHIFT)],
            out_hbm.at[rows, pl.ds(_SHIFT, T - _SHIFT)],
        )
        pltpu.sync_copy(
            x_hbm.at[rows, pl.ds(T - _SHIFT, _SHIFT)],
            out_hbm.at[rows, pl.ds(0, _SHIFT)],
        )

    return roll_k(x.reshape(R, T)).reshape(B, S, T)


# speedup vs baseline: 43.7506x; 43.7506x over previous
import functools

import jax
import jax.numpy as jnp
from jax import lax
from jax.experimental import pallas as pl
from jax.experimental.pallas import tpu as pltpu
from jax.experimental.pallas import tpu_sc as plsc

_SHIFT = 1000
_CH = 4
_NB = 3


def kernel(x):
    B, S, T = x.shape
    R = B * S
    info = plsc.get_sparse_core_info()
    NC, NS = info.num_cores, info.num_subcores
    NW = NC * NS
    rows_per_w = R // NW
    n_chunks = rows_per_w // _CH

    mesh = plsc.VectorSubcoreMesh(core_axis_name="c", subcore_axis_name="s")

    @functools.partial(
        pl.kernel,
        mesh=mesh,
        out_type=jax.ShapeDtypeStruct((R, T), jnp.float32),
        scratch_types=[pltpu.VMEM((_NB, _CH, T), jnp.float32)]
        + [pltpu.SemaphoreType.DMA] * (2 * _NB),
        compiler_params=pltpu.CompilerParams(use_tc_tiling_on_sc=False),
    )
    def roll_k(x_hbm, out_hbm, buf, *sems):
        rsem, wsem = sems[:_NB], sems[_NB:]
        wid = lax.axis_index("s") * NC + lax.axis_index("c")
        r0 = wid * rows_per_w

        def start_read(g):
            b = g % _NB
            return pltpu.async_copy(
                x_hbm.at[pl.ds(r0 + g * _CH, _CH)], buf.at[b], rsem[b]
            )

        def start_writes(g):
            b = g % _NB
            handles = []
            for i in range(_CH):
                row = r0 + g * _CH + i
                handles.append(pltpu.async_copy(
                    buf.at[b, i, pl.ds(0, T - _SHIFT)],
                    out_hbm.at[row, pl.ds(_SHIFT, T - _SHIFT)],
                    wsem[b],
                ))
                handles.append(pltpu.async_copy(
                    buf.at[b, i, pl.ds(T - _SHIFT, _SHIFT)],
                    out_hbm.at[row, pl.ds(0, _SHIFT)],
                    wsem[b],
                ))
            return handles

        reads = {}
        writes = {}
        for g in range(min(_NB, n_chunks)):
            reads[g] = start_read(g)
        for g in range(n_chunks):
            reads[g].wait()
            writes[g] = start_writes(g)
            nxt = g + _NB
            if nxt < n_chunks:
                for h in writes[g]:
                    h.wait()
                reads[nxt] = start_read(nxt)
        for g in range(max(0, n_chunks - _NB), n_chunks):
            for h in writes[g]:
                h.wait()

    return roll_k(x.reshape(R, T)).reshape(B, S, T)


# speedup vs baseline: 44.2566x; 1.0116x over previous
import functools

import jax
import jax.numpy as jnp
from jax import lax
from jax.experimental import pallas as pl
from jax.experimental.pallas import tpu as pltpu
from jax.experimental.pallas import tpu_sc as plsc

_SHIFT = 1000
_CH = 4
_NB = 3


def kernel(x):
    B, S, T = x.shape
    R = B * S
    info = plsc.get_sparse_core_info()
    NC, NS = info.num_cores, info.num_subcores
    NW = NC * NS
    rows_per_w = R // NW
    n_chunks = rows_per_w // _CH

    mesh = plsc.VectorSubcoreMesh(core_axis_name="c", subcore_axis_name="s")

    @functools.partial(
        pl.kernel,
        mesh=mesh,
        out_type=jax.ShapeDtypeStruct((R, T), jnp.float32),
        scratch_types=[pltpu.VMEM((_NB, _CH, T), jnp.float32)]
        + [pltpu.SemaphoreType.DMA] * (2 * _NB),
        compiler_params=pltpu.CompilerParams(use_tc_tiling_on_sc=False),
    )
    def roll_k(x_hbm, out_hbm, buf, *sems):
        rsem, wsem = sems[:_NB], sems[_NB:]
        wid = lax.axis_index("s") * NC + lax.axis_index("c")
        r0 = wid * rows_per_w

        def start_read(g):
            b = g % _NB
            return pltpu.async_copy(
                x_hbm.at[pl.ds(r0 + g * _CH, _CH)], buf.at[b], rsem[b]
            )

        def start_writes(g):
            b = g % _NB
            rows = pl.ds(r0 + g * _CH, _CH)
            return [
                pltpu.async_copy(
                    buf.at[b, :, pl.ds(0, T - _SHIFT)],
                    out_hbm.at[rows, pl.ds(_SHIFT, T - _SHIFT)],
                    wsem[b],
                ),
                pltpu.async_copy(
                    buf.at[b, :, pl.ds(T - _SHIFT, _SHIFT)],
                    out_hbm.at[rows, pl.ds(0, _SHIFT)],
                    wsem[b],
                ),
            ]

        reads = {}
        writes = {}
        for g in range(min(_NB, n_chunks)):
            reads[g] = start_read(g)
        for g in range(n_chunks):
            reads[g].wait()
            writes[g] = start_writes(g)
            nxt = g + _NB
            if nxt < n_chunks:
                for h in writes[g]:
                    h.wait()
                reads[nxt] = start_read(nxt)
        for g in range(max(0, n_chunks - _NB), n_chunks):
            for h in writes[g]:
                h.wait()

    return roll_k(x.reshape(R, T)).reshape(B, S, T)
